# Initial kernel scaffold; baseline (speedup 1.0000x reference)
import jax
import jax.numpy as jnp
from jax.experimental import pallas as pl


def kernel(cls_l0, cls_l1, cls_l2, cls_l3, cls_l4, reg_l0, reg_l1, reg_l2, reg_l3, reg_l4, image_size):
    raise NotImplementedError("write your pallas kernel here")


# speedup vs baseline: 9.9133x; 9.9133x over previous
import numpy as np
import jax
import jax.numpy as jnp
from jax.experimental import pallas as pl

_STRIDES = [4, 8, 16, 32, 64]
_RATIOS = [0.5, 1.0, 2.0]
_OCTAVE = 8.0
_PRE_NMS = 1000
_MAX_PER_IMG = 1000
_IOU_THR = 0.7
_MAX_RATIO = np.float32(np.abs(np.log(16.0 / 1000.0)))
_TOT = 65536
_ROWS = 512
_COLS = 128


def _anchors_np(H, W, stride):
    r = np.asarray(_RATIOS, dtype=np.float32)
    hr = np.sqrt(r).astype(np.float32)
    wr = (np.float32(1.0) / hr).astype(np.float32)
    base = np.float32(stride * _OCTAVE)
    ws = base * wr
    hs = base * hr
    ba = np.stack([-0.5 * ws, -0.5 * hs, 0.5 * ws, 0.5 * hs], axis=1)
    sx = np.arange(W, dtype=np.float32) * np.float32(stride)
    sy = np.arange(H, dtype=np.float32) * np.float32(stride)
    yy, xx = np.meshgrid(sy, sx, indexing="ij")
    shifts = np.stack([xx.ravel(), yy.ravel(), xx.ravel(), yy.ravel()], axis=1)
    return (shifts[:, None, :] + ba[None, :, :]).reshape(-1, 4).astype(np.float32)


def _nms_kernel(logit_ref, dx_ref, dy_ref, dw_ref, dh_ref,
                ax1_ref, ay1_ref, ax2_ref, ay2_ref, lev_ref, isz_ref,
                ox1_ref, oy1_ref, ox2_ref, oy2_ref, osc_ref):
    lev = lev_ref[...]
    logits = logit_ref[0]
    s = jnp.where(lev >= 0.0, jax.nn.sigmoid(logits), -1.0)

    def _thresh(lvl):
        def body(_, c):
            lo, hi = c
            mid = 0.5 * (lo + hi)
            cnt = jnp.sum(jnp.where((s >= mid) & (lev == lvl), 1.0, 0.0))
            p = cnt >= np.float32(_PRE_NMS)
            return (jnp.where(p, mid, lo), jnp.where(p, hi, mid))
        lo, _ = jax.lax.fori_loop(0, 50, body, (jnp.float32(0.0), jnp.float32(1.0)))
        return lo

    thr0 = _thresh(jnp.float32(0.0))
    thr1 = _thresh(jnp.float32(1.0))
    thr2 = _thresh(jnp.float32(2.0))
    valid0 = (((lev == 0.0) & (s >= thr0)) | ((lev == 1.0) & (s >= thr1))
              | ((lev == 2.0) & (s >= thr2)) | (lev == 3.0) | (lev == 4.0))

    isz = isz_ref[0, 0]
    ax1 = ax1_ref[...]
    ay1 = ay1_ref[...]
    ax2 = ax2_ref[...]
    ay2 = ay2_ref[...]
    aw = ax2 - ax1
    ah = ay2 - ay1
    acx = ax1 + 0.5 * aw
    acy = ay1 + 0.5 * ah
    dwc = jnp.clip(dw_ref[0], -_MAX_RATIO, _MAX_RATIO)
    dhc = jnp.clip(dh_ref[0], -_MAX_RATIO, _MAX_RATIO)
    pcx = acx + dx_ref[0] * aw
    pcy = acy + dy_ref[0] * ah
    pw = aw * jnp.exp(dwc)
    ph = ah * jnp.exp(dhc)
    x1 = jnp.clip(pcx - 0.5 * pw, 0.0, isz)
    y1 = jnp.clip(pcy - 0.5 * ph, 0.0, isz)
    x2 = jnp.clip(pcx + 0.5 * pw, 0.0, isz)
    y2 = jnp.clip(pcy + 0.5 * ph, 0.0, isz)
    off = jnp.maximum(lev, 0.0) * (isz + 1.0)
    x1o = x1 + off
    y1o = y1 + off
    x2o = x2 + off
    y2o = y2 + off
    areas = (x2o - x1o) * (y2o - y1o)

    ridx = jax.lax.broadcasted_iota(jnp.int32, (_ROWS, _COLS), 0)
    cidx = jax.lax.broadcasted_iota(jnp.int32, (_ROWS, _COLS), 1)
    idx = ridx * np.int32(_COLS) + cidx

    def nms_body(i, validf):
        alive = validf > 0.5
        m = jnp.where(alive, s, -1.0)
        gmax = jnp.max(m)
        has = gmax > -0.5
        best = jnp.min(jnp.where((m == gmax) & alive, idx, np.int32(2**30)))
        sel = idx == best

        def pick(a):
            return jnp.sum(jnp.where(sel, a, 0.0))

        bx1 = pick(x1o)
        by1 = pick(y1o)
        bx2 = pick(x2o)
        by2 = pick(y2o)
        barea = (bx2 - bx1) * (by2 - by1)
        ix1 = jnp.maximum(bx1, x1o)
        iy1 = jnp.maximum(by1, y1o)
        ix2 = jnp.minimum(bx2, x2o)
        iy2 = jnp.minimum(by2, y2o)
        inter = jnp.clip(ix2 - ix1, 0.0) * jnp.clip(iy2 - iy1, 0.0)
        iou = inter / (barea + areas - inter + 1e-9)
        keepm = jnp.where(iou <= np.float32(_IOU_THR), 1.0, 0.0)
        validf = jnp.where(has, validf * keepm, validf)

        hasf = jnp.where(has, 1.0, 0.0).astype(jnp.float32)
        ox1_ref[pl.ds(0, 1), pl.ds(i, 1), pl.ds(0, 1)] = (hasf * pick(x1)).reshape(1, 1, 1)
        oy1_ref[pl.ds(0, 1), pl.ds(i, 1), pl.ds(0, 1)] = (hasf * pick(y1)).reshape(1, 1, 1)
        ox2_ref[pl.ds(0, 1), pl.ds(i, 1), pl.ds(0, 1)] = (hasf * pick(x2)).reshape(1, 1, 1)
        oy2_ref[pl.ds(0, 1), pl.ds(i, 1), pl.ds(0, 1)] = (hasf * pick(y2)).reshape(1, 1, 1)
        osc_ref[pl.ds(0, 1), pl.ds(i, 1), pl.ds(0, 1)] = (hasf * pick(s)).reshape(1, 1, 1)
        return validf

    jax.lax.fori_loop(0, _MAX_PER_IMG, nms_body,
                      jnp.where(valid0, 1.0, 0.0).astype(jnp.float32))


def kernel(cls_l0, cls_l1, cls_l2, cls_l3, cls_l4,
           reg_l0, reg_l1, reg_l2, reg_l3, reg_l4, image_size):
    cls_list = [cls_l0, cls_l1, cls_l2, cls_l3, cls_l4]
    reg_list = [reg_l0, reg_l1, reg_l2, reg_l3, reg_l4]
    B = cls_list[0].shape[0]

    logit_parts, reg_parts, anc_parts, lev_parts = [], [], [], []
    for lvl, (c, r) in enumerate(zip(cls_list, reg_list)):
        H, W = c.shape[2], c.shape[3]
        n = H * W * 3
        logit_parts.append(jnp.transpose(c, (0, 2, 3, 1)).reshape(B, n))
        reg_parts.append(jnp.transpose(r, (0, 2, 3, 1)).reshape(B, n, 4))
        anc_parts.append(_anchors_np(H, W, _STRIDES[lvl]))
        lev_parts.append(np.full((n,), float(lvl), dtype=np.float32))

    logits = jnp.concatenate(logit_parts, axis=1)
    regs = jnp.concatenate(reg_parts, axis=1)
    ancs = np.concatenate(anc_parts, axis=0)
    levs = np.concatenate(lev_parts, axis=0)
    n_act = logits.shape[1]
    pad = _TOT - n_act
    logits = jnp.pad(logits, ((0, 0), (0, pad)))
    regs = jnp.pad(regs, ((0, 0), (0, pad), (0, 0)))
    ancs = np.pad(ancs, ((0, pad), (0, 0)))
    levs = np.pad(levs, ((0, pad),), constant_values=-1.0)

    logits = logits.reshape(B, _ROWS, _COLS)
    dx = regs[..., 0].reshape(B, _ROWS, _COLS)
    dy = regs[..., 1].reshape(B, _ROWS, _COLS)
    dw = regs[..., 2].reshape(B, _ROWS, _COLS)
    dh = regs[..., 3].reshape(B, _ROWS, _COLS)
    ax1 = jnp.asarray(ancs[:, 0].reshape(_ROWS, _COLS))
    ay1 = jnp.asarray(ancs[:, 1].reshape(_ROWS, _COLS))
    ax2 = jnp.asarray(ancs[:, 2].reshape(_ROWS, _COLS))
    ay2 = jnp.asarray(ancs[:, 3].reshape(_ROWS, _COLS))
    lev = jnp.asarray(levs.reshape(_ROWS, _COLS))
    isz = jnp.asarray(image_size, jnp.float32).reshape(1, 1)

    bspec = pl.BlockSpec((1, _ROWS, _COLS), lambda b: (b, 0, 0))
    sspec = pl.BlockSpec((_ROWS, _COLS), lambda b: (0, 0))
    ospec = pl.BlockSpec((1, _MAX_PER_IMG, 1), lambda b: (b, 0, 0))
    oshape = jax.ShapeDtypeStruct((B, _MAX_PER_IMG, 1), jnp.float32)

    outs = pl.pallas_call(
        _nms_kernel,
        grid=(B,),
        in_specs=[bspec, bspec, bspec, bspec, bspec,
                  sspec, sspec, sspec, sspec, sspec,
                  pl.BlockSpec((1, 1), lambda b: (0, 0))],
        out_specs=[ospec] * 5,
        out_shape=[oshape] * 5,
    )(logits, dx, dy, dw, dh, ax1, ay1, ax2, ay2, lev, isz)

    x1o, y1o, x2o, y2o, sc = outs
    boxes = jnp.concatenate([x1o, y1o, x2o, y2o], axis=-1)
    scores = sc[..., 0]
    return boxes, scores


# speedup vs baseline: 9.9839x; 1.0071x over previous
import numpy as np
import jax
import jax.numpy as jnp
from jax.experimental import pallas as pl
from jax.experimental.pallas import tpu as pltpu

_STRIDES = [4, 8, 16, 32, 64]
_RATIOS = [0.5, 1.0, 2.0]
_OCTAVE = 8.0
_PRE_NMS = 1000
_MAX_PER_IMG = 1000
_IOU_THR = 0.7
_MAX_RATIO = np.float32(np.abs(np.log(16.0 / 1000.0)))
_TOT = 65536
_ROWS = 512
_COLS = 128


def _anchors_np(H, W, stride):
    r = np.asarray(_RATIOS, dtype=np.float32)
    hr = np.sqrt(r).astype(np.float32)
    wr = (np.float32(1.0) / hr).astype(np.float32)
    base = np.float32(stride * _OCTAVE)
    ws = base * wr
    hs = base * hr
    ba = np.stack([-0.5 * ws, -0.5 * hs, 0.5 * ws, 0.5 * hs], axis=1)
    sx = np.arange(W, dtype=np.float32) * np.float32(stride)
    sy = np.arange(H, dtype=np.float32) * np.float32(stride)
    yy, xx = np.meshgrid(sy, sx, indexing="ij")
    shifts = np.stack([xx.ravel(), yy.ravel(), xx.ravel(), yy.ravel()], axis=1)
    return (shifts[:, None, :] + ba[None, :, :]).reshape(-1, 4).astype(np.float32)


def _nms_kernel(logit_ref, dx_ref, dy_ref, dw_ref, dh_ref,
                ax1_ref, ay1_ref, ax2_ref, ay2_ref, lev_ref, isz_ref,
                ox1_ref, oy1_ref, ox2_ref, oy2_ref, osc_ref):
    lev = lev_ref[...]
    logits = logit_ref[0]
    s = jnp.where(lev >= 0.0, jax.nn.sigmoid(logits), -1.0)

    def _thresh(lvl):
        def body(_, c):
            lo, hi = c
            mid = 0.5 * (lo + hi)
            cnt = jnp.sum(jnp.where((s >= mid) & (lev == lvl), 1.0, 0.0))
            p = cnt >= np.float32(_PRE_NMS)
            return (jnp.where(p, mid, lo), jnp.where(p, hi, mid))
        lo, _ = jax.lax.fori_loop(0, 50, body, (jnp.float32(0.0), jnp.float32(1.0)))
        return lo

    thr0 = _thresh(jnp.float32(0.0))
    thr1 = _thresh(jnp.float32(1.0))
    thr2 = _thresh(jnp.float32(2.0))
    valid0 = (((lev == 0.0) & (s >= thr0)) | ((lev == 1.0) & (s >= thr1))
              | ((lev == 2.0) & (s >= thr2)) | (lev == 3.0) | (lev == 4.0))

    isz = isz_ref[0, 0]
    ax1 = ax1_ref[...]
    ay1 = ay1_ref[...]
    ax2 = ax2_ref[...]
    ay2 = ay2_ref[...]
    aw = ax2 - ax1
    ah = ay2 - ay1
    acx = ax1 + 0.5 * aw
    acy = ay1 + 0.5 * ah
    dwc = jnp.clip(dw_ref[0], -_MAX_RATIO, _MAX_RATIO)
    dhc = jnp.clip(dh_ref[0], -_MAX_RATIO, _MAX_RATIO)
    pcx = acx + dx_ref[0] * aw
    pcy = acy + dy_ref[0] * ah
    pw = aw * jnp.exp(dwc)
    ph = ah * jnp.exp(dhc)
    x1 = jnp.clip(pcx - 0.5 * pw, 0.0, isz)
    y1 = jnp.clip(pcy - 0.5 * ph, 0.0, isz)
    x2 = jnp.clip(pcx + 0.5 * pw, 0.0, isz)
    y2 = jnp.clip(pcy + 0.5 * ph, 0.0, isz)
    off = jnp.maximum(lev, 0.0) * (isz + 1.0)
    x1o = x1 + off
    y1o = y1 + off
    x2o = x2 + off
    y2o = y2 + off
    areas = (x2o - x1o) * (y2o - y1o)

    ridx = jax.lax.broadcasted_iota(jnp.int32, (_ROWS, _COLS), 0)
    cidx = jax.lax.broadcasted_iota(jnp.int32, (_ROWS, _COLS), 1)
    idx = ridx * np.int32(_COLS) + cidx

    def nms_body(i, validf):
        alive = validf > 0.5
        m = jnp.where(alive, s, -1.0)
        gmax = jnp.max(m)
        has = gmax > -0.5
        best = jnp.min(jnp.where((m == gmax) & alive, idx, np.int32(2**30)))
        sel = idx == best

        def pick(a):
            return jnp.sum(jnp.where(sel, a, 0.0))

        bx1 = pick(x1o)
        by1 = pick(y1o)
        bx2 = pick(x2o)
        by2 = pick(y2o)
        barea = (bx2 - bx1) * (by2 - by1)
        ix1 = jnp.maximum(bx1, x1o)
        iy1 = jnp.maximum(by1, y1o)
        ix2 = jnp.minimum(bx2, x2o)
        iy2 = jnp.minimum(by2, y2o)
        inter = jnp.clip(ix2 - ix1, 0.0) * jnp.clip(iy2 - iy1, 0.0)
        iou = inter / (barea + areas - inter + 1e-9)
        keepm = jnp.where(iou <= np.float32(_IOU_THR), 1.0, 0.0)
        validf = jnp.where(has, validf * keepm, validf)

        hasf = jnp.where(has, 1.0, 0.0).astype(jnp.float32)
        ox1_ref[pl.ds(0, 1), pl.ds(i, 1), pl.ds(0, 1)] = (hasf * pick(x1)).reshape(1, 1, 1)
        oy1_ref[pl.ds(0, 1), pl.ds(i, 1), pl.ds(0, 1)] = (hasf * pick(y1)).reshape(1, 1, 1)
        ox2_ref[pl.ds(0, 1), pl.ds(i, 1), pl.ds(0, 1)] = (hasf * pick(x2)).reshape(1, 1, 1)
        oy2_ref[pl.ds(0, 1), pl.ds(i, 1), pl.ds(0, 1)] = (hasf * pick(y2)).reshape(1, 1, 1)
        osc_ref[pl.ds(0, 1), pl.ds(i, 1), pl.ds(0, 1)] = (hasf * pick(s)).reshape(1, 1, 1)
        return validf

    jax.lax.fori_loop(0, _MAX_PER_IMG, nms_body,
                      jnp.where(valid0, 1.0, 0.0).astype(jnp.float32))


def kernel(cls_l0, cls_l1, cls_l2, cls_l3, cls_l4,
           reg_l0, reg_l1, reg_l2, reg_l3, reg_l4, image_size):
    cls_list = [cls_l0, cls_l1, cls_l2, cls_l3, cls_l4]
    reg_list = [reg_l0, reg_l1, reg_l2, reg_l3, reg_l4]
    B = cls_list[0].shape[0]

    logit_parts, reg_parts, anc_parts, lev_parts = [], [], [], []
    for lvl, (c, r) in enumerate(zip(cls_list, reg_list)):
        H, W = c.shape[2], c.shape[3]
        n = H * W * 3
        logit_parts.append(jnp.transpose(c, (0, 2, 3, 1)).reshape(B, n))
        reg_parts.append(jnp.transpose(r, (0, 2, 3, 1)).reshape(B, n, 4))
        anc_parts.append(_anchors_np(H, W, _STRIDES[lvl]))
        lev_parts.append(np.full((n,), float(lvl), dtype=np.float32))

    logits = jnp.concatenate(logit_parts, axis=1)
    regs = jnp.concatenate(reg_parts, axis=1)
    ancs = np.concatenate(anc_parts, axis=0)
    levs = np.concatenate(lev_parts, axis=0)
    n_act = logits.shape[1]
    pad = _TOT - n_act
    logits = jnp.pad(logits, ((0, 0), (0, pad)))
    regs = jnp.pad(regs, ((0, 0), (0, pad), (0, 0)))
    ancs = np.pad(ancs, ((0, pad), (0, 0)))
    levs = np.pad(levs, ((0, pad),), constant_values=-1.0)

    logits = logits.reshape(B, _ROWS, _COLS)
    dx = regs[..., 0].reshape(B, _ROWS, _COLS)
    dy = regs[..., 1].reshape(B, _ROWS, _COLS)
    dw = regs[..., 2].reshape(B, _ROWS, _COLS)
    dh = regs[..., 3].reshape(B, _ROWS, _COLS)
    ax1 = jnp.asarray(ancs[:, 0].reshape(_ROWS, _COLS))
    ay1 = jnp.asarray(ancs[:, 1].reshape(_ROWS, _COLS))
    ax2 = jnp.asarray(ancs[:, 2].reshape(_ROWS, _COLS))
    ay2 = jnp.asarray(ancs[:, 3].reshape(_ROWS, _COLS))
    lev = jnp.asarray(levs.reshape(_ROWS, _COLS))
    isz = jnp.asarray(image_size, jnp.float32).reshape(1, 1)

    bspec = pl.BlockSpec((1, _ROWS, _COLS), lambda b: (b, 0, 0))
    sspec = pl.BlockSpec((_ROWS, _COLS), lambda b: (0, 0))
    ospec = pl.BlockSpec((1, _MAX_PER_IMG, 1), lambda b: (b, 0, 0))
    oshape = jax.ShapeDtypeStruct((B, _MAX_PER_IMG, 1), jnp.float32)

    outs = pl.pallas_call(
        _nms_kernel,
        grid=(B,),
        in_specs=[bspec, bspec, bspec, bspec, bspec,
                  sspec, sspec, sspec, sspec, sspec,
                  pl.BlockSpec((1, 1), lambda b: (0, 0))],
        out_specs=[ospec] * 5,
        out_shape=[oshape] * 5,
        compiler_params=pltpu.CompilerParams(
            dimension_semantics=("parallel",)),
    )(logits, dx, dy, dw, dh, ax1, ay1, ax2, ay2, lev, isz)

    x1o, y1o, x2o, y2o, sc = outs
    boxes = jnp.concatenate([x1o, y1o, x2o, y2o], axis=-1)
    scores = sc[..., 0]
    return boxes, scores


# speedup vs baseline: 10.6171x; 1.0634x over previous
import numpy as np
import jax
import jax.numpy as jnp
from jax.experimental import pallas as pl
from jax.experimental.pallas import tpu as pltpu

_STRIDES = [4, 8, 16, 32, 64]
_RATIOS = [0.5, 1.0, 2.0]
_OCTAVE = 8.0
_PRE_NMS = 1000
_MAX_PER_IMG = 1000
_IOU_THR = 0.7
_MAX_RATIO = np.float32(np.abs(np.log(16.0 / 1000.0)))
_TOT = 65536
_ROWS = 512
_COLS = 128


def _anchors_np(H, W, stride):
    r = np.asarray(_RATIOS, dtype=np.float32)
    hr = np.sqrt(r).astype(np.float32)
    wr = (np.float32(1.0) / hr).astype(np.float32)
    base = np.float32(stride * _OCTAVE)
    ws = base * wr
    hs = base * hr
    ba = np.stack([-0.5 * ws, -0.5 * hs, 0.5 * ws, 0.5 * hs], axis=1)
    sx = np.arange(W, dtype=np.float32) * np.float32(stride)
    sy = np.arange(H, dtype=np.float32) * np.float32(stride)
    yy, xx = np.meshgrid(sy, sx, indexing="ij")
    shifts = np.stack([xx.ravel(), yy.ravel(), xx.ravel(), yy.ravel()], axis=1)
    return (shifts[:, None, :] + ba[None, :, :]).reshape(-1, 4).astype(np.float32)


def _nms_kernel(logit_ref, dx_ref, dy_ref, dw_ref, dh_ref,
                ax1_ref, ay1_ref, ax2_ref, ay2_ref, lev_ref, isz_ref,
                ox1_ref, oy1_ref, ox2_ref, oy2_ref, osc_ref):
    lev = lev_ref[...]
    logits = logit_ref[0]
    s = jnp.where(lev >= 0.0, jax.nn.sigmoid(logits), -1.0)

    def _thresh(lvl):
        def body(_, c):
            lo, hi = c
            mid = 0.5 * (lo + hi)
            cnt = jnp.sum(jnp.where((s >= mid) & (lev == lvl), 1.0, 0.0))
            p = cnt >= np.float32(_PRE_NMS)
            return (jnp.where(p, mid, lo), jnp.where(p, hi, mid))
        lo, _ = jax.lax.fori_loop(0, 50, body, (jnp.float32(0.0), jnp.float32(1.0)))
        return lo

    thr0 = _thresh(jnp.float32(0.0))
    thr1 = _thresh(jnp.float32(1.0))
    thr2 = _thresh(jnp.float32(2.0))
    valid0 = (((lev == 0.0) & (s >= thr0)) | ((lev == 1.0) & (s >= thr1))
              | ((lev == 2.0) & (s >= thr2)) | (lev == 3.0) | (lev == 4.0))

    isz = isz_ref[0, 0]
    ax1 = ax1_ref[...]
    ay1 = ay1_ref[...]
    ax2 = ax2_ref[...]
    ay2 = ay2_ref[...]
    aw = ax2 - ax1
    ah = ay2 - ay1
    acx = ax1 + 0.5 * aw
    acy = ay1 + 0.5 * ah
    dwc = jnp.clip(dw_ref[0], -_MAX_RATIO, _MAX_RATIO)
    dhc = jnp.clip(dh_ref[0], -_MAX_RATIO, _MAX_RATIO)
    pcx = acx + dx_ref[0] * aw
    pcy = acy + dy_ref[0] * ah
    pw = aw * jnp.exp(dwc)
    ph = ah * jnp.exp(dhc)
    x1 = jnp.clip(pcx - 0.5 * pw, 0.0, isz)
    y1 = jnp.clip(pcy - 0.5 * ph, 0.0, isz)
    x2 = jnp.clip(pcx + 0.5 * pw, 0.0, isz)
    y2 = jnp.clip(pcy + 0.5 * ph, 0.0, isz)
    off = jnp.maximum(lev, 0.0) * (isz + 1.0)
    x1o = x1 + off
    y1o = y1 + off
    x2o = x2 + off
    y2o = y2 + off
    areas = (x2o - x1o) * (y2o - y1o)

    ridx = jax.lax.broadcasted_iota(jnp.int32, (_ROWS, _COLS), 0)
    cidx = jax.lax.broadcasted_iota(jnp.int32, (_ROWS, _COLS), 1)
    idx = ridx * np.int32(_COLS) + cidx

    def nms_body(i, validf):
        alive = validf > 0.5
        m = jnp.where(alive, s, -1.0)
        gmax = jnp.max(m)
        has = gmax > -0.5
        best = jnp.min(jnp.where((m == gmax) & alive, idx, np.int32(2**30)))
        sel = idx == best

        def pick(a):
            return jnp.sum(jnp.where(sel, a, 0.0))

        bx1 = pick(x1o)
        by1 = pick(y1o)
        bx2 = pick(x2o)
        by2 = pick(y2o)
        bsc = pick(s)
        denom = isz + 1.0
        loff = jnp.floor(bx1 / denom) * denom
        barea = (bx2 - bx1) * (by2 - by1)
        ix1 = jnp.maximum(bx1, x1o)
        iy1 = jnp.maximum(by1, y1o)
        ix2 = jnp.minimum(bx2, x2o)
        iy2 = jnp.minimum(by2, y2o)
        inter = jnp.clip(ix2 - ix1, 0.0) * jnp.clip(iy2 - iy1, 0.0)
        iou = inter / (barea + areas - inter + 1e-9)
        keepm = jnp.where(iou <= np.float32(_IOU_THR), 1.0, 0.0)
        validf = jnp.where(has, validf * keepm, validf)

        hasf = jnp.where(has, 1.0, 0.0).astype(jnp.float32)
        ox1_ref[pl.ds(0, 1), pl.ds(i, 1), pl.ds(0, 1)] = (hasf * (bx1 - loff)).reshape(1, 1, 1)
        oy1_ref[pl.ds(0, 1), pl.ds(i, 1), pl.ds(0, 1)] = (hasf * (by1 - loff)).reshape(1, 1, 1)
        ox2_ref[pl.ds(0, 1), pl.ds(i, 1), pl.ds(0, 1)] = (hasf * (bx2 - loff)).reshape(1, 1, 1)
        oy2_ref[pl.ds(0, 1), pl.ds(i, 1), pl.ds(0, 1)] = (hasf * (by2 - loff)).reshape(1, 1, 1)
        osc_ref[pl.ds(0, 1), pl.ds(i, 1), pl.ds(0, 1)] = (hasf * bsc).reshape(1, 1, 1)
        return validf

    jax.lax.fori_loop(0, _MAX_PER_IMG, nms_body,
                      jnp.where(valid0, 1.0, 0.0).astype(jnp.float32))


def kernel(cls_l0, cls_l1, cls_l2, cls_l3, cls_l4,
           reg_l0, reg_l1, reg_l2, reg_l3, reg_l4, image_size):
    cls_list = [cls_l0, cls_l1, cls_l2, cls_l3, cls_l4]
    reg_list = [reg_l0, reg_l1, reg_l2, reg_l3, reg_l4]
    B = cls_list[0].shape[0]

    logit_parts, reg_parts, anc_parts, lev_parts = [], [], [], []
    for lvl, (c, r) in enumerate(zip(cls_list, reg_list)):
        H, W = c.shape[2], c.shape[3]
        n = H * W * 3
        logit_parts.append(jnp.transpose(c, (0, 2, 3, 1)).reshape(B, n))
        reg_parts.append(jnp.transpose(r, (0, 2, 3, 1)).reshape(B, n, 4))
        anc_parts.append(_anchors_np(H, W, _STRIDES[lvl]))
        lev_parts.append(np.full((n,), float(lvl), dtype=np.float32))

    logits = jnp.concatenate(logit_parts, axis=1)
    regs = jnp.concatenate(reg_parts, axis=1)
    ancs = np.concatenate(anc_parts, axis=0)
    levs = np.concatenate(lev_parts, axis=0)
    n_act = logits.shape[1]
    pad = _TOT - n_act
    logits = jnp.pad(logits, ((0, 0), (0, pad)))
    regs = jnp.pad(regs, ((0, 0), (0, pad), (0, 0)))
    ancs = np.pad(ancs, ((0, pad), (0, 0)))
    levs = np.pad(levs, ((0, pad),), constant_values=-1.0)

    logits = logits.reshape(B, _ROWS, _COLS)
    dx = regs[..., 0].reshape(B, _ROWS, _COLS)
    dy = regs[..., 1].reshape(B, _ROWS, _COLS)
    dw = regs[..., 2].reshape(B, _ROWS, _COLS)
    dh = regs[..., 3].reshape(B, _ROWS, _COLS)
    ax1 = jnp.asarray(ancs[:, 0].reshape(_ROWS, _COLS))
    ay1 = jnp.asarray(ancs[:, 1].reshape(_ROWS, _COLS))
    ax2 = jnp.asarray(ancs[:, 2].reshape(_ROWS, _COLS))
    ay2 = jnp.asarray(ancs[:, 3].reshape(_ROWS, _COLS))
    lev = jnp.asarray(levs.reshape(_ROWS, _COLS))
    isz = jnp.asarray(image_size, jnp.float32).reshape(1, 1)

    bspec = pl.BlockSpec((1, _ROWS, _COLS), lambda b: (b, 0, 0))
    sspec = pl.BlockSpec((_ROWS, _COLS), lambda b: (0, 0))
    ospec = pl.BlockSpec((1, _MAX_PER_IMG, 1), lambda b: (b, 0, 0))
    oshape = jax.ShapeDtypeStruct((B, _MAX_PER_IMG, 1), jnp.float32)

    outs = pl.pallas_call(
        _nms_kernel,
        grid=(B,),
        in_specs=[bspec, bspec, bspec, bspec, bspec,
                  sspec, sspec, sspec, sspec, sspec,
                  pl.BlockSpec((1, 1), lambda b: (0, 0))],
        out_specs=[ospec] * 5,
        out_shape=[oshape] * 5,
        compiler_params=pltpu.CompilerParams(
            dimension_semantics=("parallel",)),
    )(logits, dx, dy, dw, dh, ax1, ay1, ax2, ay2, lev, isz)

    x1o, y1o, x2o, y2o, sc = outs
    boxes = jnp.concatenate([x1o, y1o, x2o, y2o], axis=-1)
    scores = sc[..., 0]
    return boxes, scores
